# Initial kernel scaffold; baseline (speedup 1.0000x reference)
import jax
import jax.numpy as jnp
from jax.experimental import pallas as pl


def kernel(x, weight, bias):
    raise NotImplementedError("write your pallas kernel here")


# speedup vs baseline: 7.8518x; 7.8518x over previous
import jax
import jax.numpy as jnp
from jax.experimental import pallas as pl
from jax.experimental.pallas import tpu as pltpu


def _round_up(x, m):
    return (x + m - 1) // m * m


def _linear_kernel(x_ref, w_ref, b_ref, o_ref):
    acc = jax.lax.dot_general(
        x_ref[...], w_ref[...],
        dimension_numbers=(((1,), (1,)), ((), ())),
        preferred_element_type=jnp.float32,
    )
    o_ref[...] = acc + b_ref[...]


def kernel(x, weight, bias):
    M, K = x.shape
    N, K2 = weight.shape
    assert K == K2
    out_dtype = x.dtype

    bm = min(_round_up(M, 8), 1024)
    bn = min(_round_up(N, 128), 1024)
    Kp = _round_up(K, 128)
    Mp = _round_up(M, bm)
    Np = _round_up(N, bn)

    if (Mp, Kp) != (M, K):
        x = jnp.pad(x, ((0, Mp - M), (0, Kp - K)))
    if (Np, Kp) != (N, K):
        weight = jnp.pad(weight, ((0, Np - N), (0, Kp - K)))
    if Np != N:
        bias = jnp.pad(bias, (0, Np - N))

    xb = x.astype(jnp.bfloat16)
    wb = weight.astype(jnp.bfloat16)
    b2d = bias.reshape(1, Np).astype(jnp.float32)

    cost = pl.CostEstimate(
        flops=2 * Mp * Np * Kp,
        transcendentals=0,
        bytes_accessed=2 * (Mp * Kp + Kp * Np) + 4 * Mp * Np,
    )

    out = pl.pallas_call(
        _linear_kernel,
        out_shape=jax.ShapeDtypeStruct((Mp, Np), out_dtype),
        grid=(Mp // bm, Np // bn),
        in_specs=[
            pl.BlockSpec((bm, Kp), lambda i, j: (i, 0)),
            pl.BlockSpec((bn, Kp), lambda i, j: (j, 0)),
            pl.BlockSpec((1, bn), lambda i, j: (0, j)),
        ],
        out_specs=pl.BlockSpec((bm, bn), lambda i, j: (i, j)),
        compiler_params=pltpu.CompilerParams(
            dimension_semantics=("parallel", "parallel")
        ),
        cost_estimate=cost,
    )(xb, wb, b2d)

    if (Mp, Np) != (M, N):
        out = out[:M, :N]
    return out


# speedup vs baseline: 9.7712x; 1.2445x over previous
import jax
import jax.numpy as jnp
from jax.experimental import pallas as pl
from jax.experimental.pallas import tpu as pltpu


def _round_up(x, m):
    return (x + m - 1) // m * m


def _linear_kernel(x_ref, w_ref, b_ref, o_ref):
    k = pl.program_id(2)

    @pl.when(k == 0)
    def _():
        o_ref[...] = jnp.broadcast_to(b_ref[...], o_ref.shape)

    o_ref[...] += jax.lax.dot_general(
        x_ref[...].astype(jnp.bfloat16),
        w_ref[...].astype(jnp.bfloat16),
        dimension_numbers=(((1,), (1,)), ((), ())),
        preferred_element_type=jnp.float32,
    )


def kernel(x, weight, bias):
    M, K = x.shape
    N, K2 = weight.shape
    assert K == K2
    out_dtype = x.dtype

    bm = min(_round_up(M, 8), 2048)
    bn = min(_round_up(N, 128), 2048)
    bk = min(_round_up(K, 128), 512)
    Mp = _round_up(M, bm)
    Np = _round_up(N, bn)
    Kp = _round_up(K, bk)

    if (Mp, Kp) != (M, K):
        x = jnp.pad(x, ((0, Mp - M), (0, Kp - K)))
    if (Np, Kp) != (N, K):
        weight = jnp.pad(weight, ((0, Np - N), (0, Kp - K)))
    if Np != N:
        bias = jnp.pad(bias, (0, Np - N))
    b2d = bias.reshape(1, Np).astype(jnp.float32)

    cost = pl.CostEstimate(
        flops=2 * Mp * Np * Kp,
        transcendentals=0,
        bytes_accessed=4 * (2 * Mp * Kp + 2 * Kp * Np + Mp * Np),
    )

    out = pl.pallas_call(
        _linear_kernel,
        out_shape=jax.ShapeDtypeStruct((Mp, Np), out_dtype),
        grid=(Mp // bm, Np // bn, Kp // bk),
        in_specs=[
            pl.BlockSpec((bm, bk), lambda i, j, k: (i, k)),
            pl.BlockSpec((bn, bk), lambda i, j, k: (j, k)),
            pl.BlockSpec((1, bn), lambda i, j, k: (0, j)),
        ],
        out_specs=pl.BlockSpec((bm, bn), lambda i, j, k: (i, j)),
        compiler_params=pltpu.CompilerParams(
            dimension_semantics=("parallel", "parallel", "arbitrary")
        ),
        cost_estimate=cost,
    )(x, weight, b2d)

    if (Mp, Np) != (M, N):
        out = out[:M, :N]
    return out
